# baseline (speedup 1.0000x reference)
import jax
import jax.numpy as jnp
import numpy as np
from jax import lax
from jax.experimental import pallas as pl
from jax.experimental.pallas import tpu as pltpu
from jax.experimental.pallas import tpu_sc as plsc

N_NODES = 25000
D_IN = 128
D_OUT = 128
NSUP = 4
DS = D_OUT // NSUP
E = 160000
N_TILES = 16
CHUNK = 128
CHUNKS = 80
E_PAD = N_TILES * CHUNKS * CHUNK
ACC_ROWS = 25088
TILE_ROWS = ACC_ROWS // N_TILES
RB = 112
RB_ITERS = TILE_ROWS // RB
ZB = 56
Z_ITERS = TILE_ROWS // ZB

_PERM = np.concatenate(
    [32 * i + np.arange(32).reshape(16, 2).T.reshape(-1).argsort()
     for i in range(NSUP)])


def _project_body(xu_ref, xv_ref, w_ref, *out_refs):
    w = w_ref[...]
    hu = jnp.dot(xu_ref[...], w, preferred_element_type=jnp.float32)
    hv = jnp.dot(xv_ref[...], w, preferred_element_type=jnp.float32)
    for i in range(NSUP):
        out_refs[i][...] = hu[:, i * DS:(i + 1) * DS].astype(jnp.bfloat16)
        out_refs[NSUP + i][...] = hv[:, i * DS:(i + 1) * DS].astype(jnp.bfloat16)


def _project(x_u, x_v, W):
    rb = 1568
    return pl.pallas_call(
        _project_body,
        grid=(ACC_ROWS // rb,),
        in_specs=[
            pl.BlockSpec((rb, D_IN), lambda r: (r, 0)),
            pl.BlockSpec((rb, D_IN), lambda r: (r, 0)),
            pl.BlockSpec((D_IN, D_OUT), lambda r: (0, 0)),
        ],
        out_specs=[pl.BlockSpec((rb, DS), lambda r: (r, 0))] * (2 * NSUP),
        out_shape=[jax.ShapeDtypeStruct((ACC_ROWS, DS), jnp.bfloat16)] * (2 * NSUP),

    )(x_u, x_v, W)


def _sc_body(tu0, tu1, tu2, tu3, tv0, tv1, tv2, tv3,
             eu, ev, val, valt, out_u, out_v,
             acc, tbl_s, zbuf, rows, rows_b, rows_c, rows_d, sbuf, sbuf_b,
             rbuf, isrc, idst, vbuf,
             gsem, gsem_b, gsem_c, gsem_d, ssem, ssem_b):
    c = lax.axis_index("c")
    s = lax.axis_index("s")
    t_u = [tu0, tu1, tu2, tu3]
    t_v = [tv0, tv1, tv2, tv3]
    base = s * TILE_ROWS

    @pl.loop(0, ZB)
    def _zinit(r):
        for h in range(2):
            zbuf[r, pl.ds(16 * h, 16)] = jnp.zeros((16,), jnp.float32)

    def load_phase(tbl, e_src, e_dst, e_val):
      with jax.named_scope("load_phase"):
        @pl.loop(0, Z_ITERS)
        def _zero(k):
            pltpu.sync_copy(zbuf, acc.at[pl.ds(base + k * ZB, ZB)])
        pltpu.sync_copy(tbl.at[pl.ds(base, TILE_ROWS)],
                        tbl_s.at[pl.ds(base, TILE_ROWS)])
        pltpu.sync_copy(e_src, isrc)
        pltpu.sync_copy(e_dst, idst)
        pltpu.sync_copy(e_val, vbuf)

    def edge_phase():
      with jax.named_scope("edge_phase"):
        bufs = (rows, rows_b, rows_c, rows_d)
        gsems = (gsem, gsem_b, gsem_c, gsem_d)
        sbufs = (sbuf, sbuf_b)
        ssems = (ssem, ssem_b)
        for b in range(2):
            pltpu.async_copy(tbl_s.at[isrc.at[b]], bufs[b], gsems[b])

        @pl.loop(0, CHUNKS, step=4)
        def _chunk(j):
            for b in range(4):
                jj = j + b
                sb = b % 2
                pltpu.make_async_copy(tbl_s.at[isrc.at[jj]], bufs[b],
                                      gsems[b]).wait()

                @pl.when(jj >= 2)
                def _():
                    pltpu.make_async_copy(
                        sbufs[sb], acc.at[idst.at[jj - 2]], ssems[sb]).wait()

                @plsc.parallel_loop(0, CHUNK // 16, unroll=2)
                def _scale(g):
                    vv = vbuf[jj, pl.ds(g * 16, 16)]
                    for t in range(16):
                        e = g * 16 + t
                        v = lax.gather(
                            vv, jnp.full((16, 1), t, jnp.int32),
                            lax.GatherDimensionNumbers(
                                offset_dims=(), collapsed_slice_dims=(0,),
                                start_index_map=(0,)),
                            slice_sizes=(1,),
                            mode=lax.GatherScatterMode.PROMISE_IN_BOUNDS)
                        bf = plsc.bitcast(bufs[b][e, :], jnp.bfloat16)
                        lo, hi = plsc.unpack(
                            bf, format=plsc.PackFormat.INTERLEAVED)
                        sbufs[sb][e, pl.ds(0, 16)] = lo * v
                        sbufs[sb][e, pl.ds(16, 16)] = hi * v

                pltpu.async_copy(sbufs[sb], acc.at[idst.at[jj]], ssems[sb],
                                 add=True)

                b2 = (b + 2) % 4

                @pl.when(jj + 2 < CHUNKS)
                def _():
                    pltpu.async_copy(tbl_s.at[isrc.at[jj + 2]], bufs[b2],
                                     gsems[b2])

        for b in range(2):
            jj = CHUNKS - 2 + b
            pltpu.make_async_copy(sbufs[jj % 2], acc.at[idst.at[jj]],
                                  ssems[jj % 2]).wait()

    def store_phase(out_ref, col):
      with jax.named_scope("store_phase"):
        @pl.loop(0, RB_ITERS)
        def _store(k):
            r0 = base + k * RB
            pltpu.sync_copy(acc.at[pl.ds(r0, RB)], rbuf)

            @pl.loop(0, RB)
            def _relu(r):
                for h in range(2):
                    sl = pl.ds(16 * h, 16)
                    rbuf[r, sl] = jnp.maximum(rbuf[r, sl], 0.0)

            pltpu.sync_copy(rbuf, out_ref.at[pl.ds(r0, RB), pl.ds(col, DS)])

    for p in range(NSUP):
        @pl.when(c == 0)
        def _():
            load_phase(t_v[p], ev.at[p, s], eu.at[p, s], val.at[p, s])

        @pl.when(c == 1)
        def _():
            load_phase(t_u[p], eu.at[p, s], ev.at[p, s], valt.at[p, s])

        plsc.subcore_barrier()

        edge_phase()

        plsc.subcore_barrier()

        @pl.when(c == 0)
        def _():
            store_phase(out_u, p * DS)

        @pl.when(c == 1)
        def _():
            store_phase(out_v, p * DS)


_SC_CALL_CACHE = []


def _sc_call(*args):
    if not _SC_CALL_CACHE:
        _SC_CALL_CACHE.append(pl.kernel(
            _sc_body,
            out_type=[jax.ShapeDtypeStruct((ACC_ROWS, D_OUT), jnp.float32)] * 2,
            mesh=plsc.VectorSubcoreMesh(core_axis_name="c", subcore_axis_name="s"),
            compiler_params=pltpu.CompilerParams(use_tc_tiling_on_sc=False,
                                                 needs_layout_passes=False),
            scratch_types=[
                pltpu.VMEM_SHARED((ACC_ROWS, DS), jnp.float32),
                pltpu.VMEM_SHARED((ACC_ROWS, DS // 2), jnp.int32),
                pltpu.VMEM((ZB, DS), jnp.float32),
                pltpu.VMEM((CHUNK, DS // 2), jnp.int32),
                pltpu.VMEM((CHUNK, DS // 2), jnp.int32),
                pltpu.VMEM((CHUNK, DS // 2), jnp.int32),
                pltpu.VMEM((CHUNK, DS // 2), jnp.int32),
                pltpu.VMEM((CHUNK, DS), jnp.float32),
                pltpu.VMEM((CHUNK, DS), jnp.float32),
                pltpu.VMEM((RB, DS), jnp.float32),
                pltpu.VMEM((CHUNKS, CHUNK), jnp.int32),
                pltpu.VMEM((CHUNKS, CHUNK), jnp.int32),
                pltpu.VMEM((CHUNKS, CHUNK), jnp.float32),
            ] + [pltpu.SemaphoreType.DMA] * 6,
        ))
    return _SC_CALL_CACHE[0](*args)


def kernel(x_u, x_v, edge_u, edge_v, edge_val, edge_val_t, W):
    xp_u = jnp.pad(x_u, ((0, ACC_ROWS - N_NODES), (0, 0)))
    xp_v = jnp.pad(x_v, ((0, ACC_ROWS - N_NODES), (0, 0)))
    tabs = _project(xp_u, xp_v, W[:, _PERM])
    pad = E_PAD - E

    def pad4(a):
        return jnp.pad(a, ((0, 0), (0, pad))).reshape(NSUP, N_TILES, CHUNKS, CHUNK)

    eu = pad4(edge_u)
    ev = pad4(edge_v)
    val = pad4(edge_val)
    valt = pad4(edge_val_t)
    tabs = [lax.bitcast_convert_type(t.reshape(ACC_ROWS, DS // 2, 2), jnp.int32)
            for t in tabs]
    out_u, out_v = _sc_call(*tabs, eu, ev, val, valt)
    return out_u[:N_NODES], out_v[:N_NODES]


# speedup vs baseline: 1.3774x; 1.3774x over previous
import jax
import jax.numpy as jnp
from jax import lax
from jax.experimental import pallas as pl
from jax.experimental.pallas import tpu as pltpu
from jax.experimental.pallas import tpu_sc as plsc

N_NODES = 25000
D_IN = 128
D_OUT = 128
NSUP = 4
DS = D_OUT // NSUP
E = 160000
N_TILES = 16
CHUNK = 128
CHUNKS = 80
QCHUNKS = 20
E_PAD = N_TILES * CHUNKS * CHUNK
ACC_ROWS = 25088
TILE_ROWS = ACC_ROWS // N_TILES
RB = 112
RB_ITERS = TILE_ROWS // RB


def _project_body(xu_ref, xv_ref, w_ref, *out_refs):
    w = w_ref[...]
    hu = jnp.dot(xu_ref[...], w, preferred_element_type=jnp.float32)
    hv = jnp.dot(xv_ref[...], w, preferred_element_type=jnp.float32)
    for i in range(NSUP):
        out_refs[i][...] = hu[:, i * DS:(i + 1) * DS]
        out_refs[NSUP + i][...] = hv[:, i * DS:(i + 1) * DS]


def _project(x_u, x_v, W):
    rb = 1568
    return pl.pallas_call(
        _project_body,
        grid=(ACC_ROWS // rb,),
        in_specs=[
            pl.BlockSpec((rb, D_IN), lambda r: (r, 0)),
            pl.BlockSpec((rb, D_IN), lambda r: (r, 0)),
            pl.BlockSpec((D_IN, D_OUT), lambda r: (0, 0)),
        ],
        out_specs=[pl.BlockSpec((rb, DS), lambda r: (r, 0))] * (2 * NSUP),
        out_shape=[jax.ShapeDtypeStruct((ACC_ROWS, DS), jnp.float32)] * (2 * NSUP),
    )(x_u, x_v, W)


def _zero_rbuf(rbuf):
    @pl.loop(0, RB)
    def _z(r):
        for h in range(2):
            rbuf[r, pl.ds(16 * h, 16)] = jnp.zeros((16,), jnp.float32)


def _sc_body(tu0, tu1, tu2, tu3, tv0, tv1, tv2, tv3,
             esrc, edst, evals, out_u, out_v,
             acc, tbl_s, rows, rows_b, rows_c, rows_d, rbuf, isrc, idst, vbuf,
             gsem, gsem_b, gsem_c, gsem_d, ssem, ssem_b, ssem_c, ssem_d):
    c = lax.axis_index("c")
    s = lax.axis_index("s")
    t_u = [tu0, tu1, tu2, tu3]
    t_v = [tv0, tv1, tv2, tv3]
    base = s * TILE_ROWS

    _zero_rbuf(rbuf)

    def load_phase(tbl):
      with jax.named_scope("load_phase"):
        @pl.loop(0, RB_ITERS)
        def _zero(k):
            pltpu.sync_copy(rbuf, acc.at[pl.ds(base + k * RB, RB)])

        pltpu.sync_copy(tbl.at[pl.ds(base, TILE_ROWS)],
                        tbl_s.at[pl.ds(base, TILE_ROWS)])

    def edge_phase(p):
      with jax.named_scope("edge_phase"):
        bufs = (rows, rows_b, rows_c, rows_d)
        gsems = (gsem, gsem_b, gsem_c, gsem_d)
        ssems = (ssem, ssem_b, ssem_c, ssem_d)
        e_src = esrc.at[c, p, s]
        e_dst = edst.at[c, p, s]
        e_val = evals.at[c, p, s]

        @pl.loop(0, CHUNKS // QCHUNKS)
        def _quarter(q):
            pltpu.sync_copy(e_src.at[pl.ds(q * QCHUNKS, QCHUNKS)], isrc)
            pltpu.sync_copy(e_dst.at[pl.ds(q * QCHUNKS, QCHUNKS)], idst)
            pltpu.sync_copy(e_val.at[pl.ds(q * QCHUNKS, QCHUNKS)], vbuf)
            for b in range(2):
                pltpu.async_copy(tbl_s.at[isrc.at[b]], bufs[b], gsems[b])

            @pl.loop(0, QCHUNKS, step=4)
            def _chunk(j):
                for b in range(4):
                    jj = j + b
                    pltpu.make_async_copy(tbl_s.at[isrc.at[jj]], bufs[b],
                                          gsems[b]).wait()

                    @plsc.parallel_loop(0, CHUNK // 16)
                    def _scale(g):
                        vv = vbuf[jj, pl.ds(g * 16, 16)]
                        for t in range(16):
                            e = g * 16 + t
                            v = lax.gather(
                                vv, jnp.full((16, 1), t, jnp.int32),
                                lax.GatherDimensionNumbers(
                                    offset_dims=(), collapsed_slice_dims=(0,),
                                    start_index_map=(0,)),
                                slice_sizes=(1,),
                                mode=lax.GatherScatterMode.PROMISE_IN_BOUNDS)
                            for h in range(2):
                                sl = pl.ds(16 * h, 16)
                                bufs[b][e, sl] = bufs[b][e, sl] * v

                    pltpu.async_copy(bufs[b], acc.at[idst.at[jj]], ssems[b],
                                     add=True)

                    b2 = (b + 2) % 4

                    @pl.when(jj + 2 < QCHUNKS)
                    def _():
                        @pl.when(jj >= 2)
                        def _():
                            pltpu.make_async_copy(
                                bufs[b2], acc.at[idst.at[jj - 2]],
                                ssems[b2]).wait()

                        pltpu.async_copy(tbl_s.at[isrc.at[jj + 2]], bufs[b2],
                                         gsems[b2])

            for b in range(4):
                jj = QCHUNKS - 4 + b
                pltpu.make_async_copy(bufs[b], acc.at[idst.at[jj]],
                                      ssems[b]).wait()
            return None

    def store_phase(out_ref, col):
      with jax.named_scope("store_phase"):
        @pl.loop(0, RB_ITERS)
        def _store(k):
            r0 = base + k * RB
            pltpu.sync_copy(acc.at[pl.ds(r0, RB)], rbuf)

            @pl.loop(0, RB)
            def _relu(r):
                for h in range(2):
                    sl = pl.ds(16 * h, 16)
                    rbuf[r, sl] = jnp.maximum(rbuf[r, sl], 0.0)

            pltpu.sync_copy(rbuf, out_ref.at[pl.ds(r0, RB), pl.ds(col, DS)])

        _zero_rbuf(rbuf)

    for p in range(NSUP):
        @pl.when(c == 0)
        def _():
            load_phase(t_v[p])

        @pl.when(c == 1)
        def _():
            load_phase(t_u[p])

        plsc.subcore_barrier()

        edge_phase(p)

        plsc.subcore_barrier()

        @pl.when(c == 0)
        def _():
            store_phase(out_u, p * DS)

        @pl.when(c == 1)
        def _():
            store_phase(out_v, p * DS)


_SC_CALL_CACHE = []


def _sc_call(*args):
    if not _SC_CALL_CACHE:
        _SC_CALL_CACHE.append(pl.kernel(
            _sc_body,
            out_type=[jax.ShapeDtypeStruct((ACC_ROWS, D_OUT), jnp.float32)] * 2,
            mesh=plsc.VectorSubcoreMesh(core_axis_name="c", subcore_axis_name="s"),
            compiler_params=pltpu.CompilerParams(use_tc_tiling_on_sc=False),
            scratch_types=[
                pltpu.VMEM_SHARED((ACC_ROWS, DS), jnp.float32),
                pltpu.VMEM_SHARED((ACC_ROWS, DS), jnp.float32),
                pltpu.VMEM((CHUNK, DS), jnp.float32),
                pltpu.VMEM((CHUNK, DS), jnp.float32),
                pltpu.VMEM((CHUNK, DS), jnp.float32),
                pltpu.VMEM((CHUNK, DS), jnp.float32),
                pltpu.VMEM((RB, DS), jnp.float32),
                pltpu.VMEM((QCHUNKS, CHUNK), jnp.int32),
                pltpu.VMEM((QCHUNKS, CHUNK), jnp.int32),
                pltpu.VMEM((QCHUNKS, CHUNK), jnp.float32),
            ] + [pltpu.SemaphoreType.DMA] * 8,
        ))
    return _SC_CALL_CACHE[0](*args)


def kernel(x_u, x_v, edge_u, edge_v, edge_val, edge_val_t, W):
    xp_u = jnp.pad(x_u, ((0, ACC_ROWS - N_NODES), (0, 0)))
    xp_v = jnp.pad(x_v, ((0, ACC_ROWS - N_NODES), (0, 0)))
    tabs = _project(xp_u, xp_v, W)
    pad = E_PAD - E

    def pad4(a):
        return jnp.pad(a, ((0, 0), (0, pad))).reshape(NSUP, N_TILES, CHUNKS, CHUNK)

    eu = pad4(edge_u)
    ev = pad4(edge_v)
    val = pad4(edge_val)
    valt = pad4(edge_val_t)
    esrc = jnp.stack([ev, eu])
    edst = jnp.stack([eu, ev])
    evals = jnp.stack([val, valt])
    out_u, out_v = _sc_call(*tabs, esrc, edst, evals)
    return out_u[:N_NODES], out_v[:N_NODES]


# speedup vs baseline: 1.4405x; 1.0458x over previous
import jax
import jax.numpy as jnp
from jax import lax
from jax.experimental import pallas as pl
from jax.experimental.pallas import tpu as pltpu
from jax.experimental.pallas import tpu_sc as plsc

N_NODES = 25000
D_IN = 128
D_OUT = 128
NSUP = 4
DS = D_OUT // NSUP
E = 160000
N_TILES = 16
CHUNK = 128
CHUNKS = 80
QCHUNKS = 20
E_PAD = N_TILES * CHUNKS * CHUNK
ACC_ROWS = 25088
TILE_ROWS = ACC_ROWS // N_TILES
RB = 112
RB_ITERS = TILE_ROWS // RB


def _project_body(xu_ref, xv_ref, w_ref, *out_refs):
    w = w_ref[...]
    hu = jnp.dot(xu_ref[...], w, preferred_element_type=jnp.float32)
    hv = jnp.dot(xv_ref[...], w, preferred_element_type=jnp.float32)
    for i in range(NSUP):
        out_refs[i][...] = hu[:, i * DS:(i + 1) * DS]
        out_refs[NSUP + i][...] = hv[:, i * DS:(i + 1) * DS]


def _project(x_u, x_v, W):
    rb = 1568
    return pl.pallas_call(
        _project_body,
        grid=(ACC_ROWS // rb,),
        in_specs=[
            pl.BlockSpec((rb, D_IN), lambda r: (r, 0)),
            pl.BlockSpec((rb, D_IN), lambda r: (r, 0)),
            pl.BlockSpec((D_IN, D_OUT), lambda r: (0, 0)),
        ],
        out_specs=[pl.BlockSpec((rb, DS), lambda r: (r, 0))] * (2 * NSUP),
        out_shape=[jax.ShapeDtypeStruct((ACC_ROWS, DS), jnp.float32)] * (2 * NSUP),
    )(x_u, x_v, W)


def _zero_rbuf(rbuf):
    @pl.loop(0, RB)
    def _z(r):
        for h in range(2):
            rbuf[r, pl.ds(16 * h, 16)] = jnp.zeros((16,), jnp.float32)


def _sc_body(tu0, tu1, tu2, tu3, tv0, tv1, tv2, tv3,
             eu, ev, val, valt, out_u, out_v,
             acc, tbl_s, rows, rows_b, rows_c, rows_d, rbuf, isrc, idst, vbuf,
             gsem, gsem_b, gsem_c, gsem_d, ssem, ssem_b, ssem_c, ssem_d):
    c = lax.axis_index("c")
    s = lax.axis_index("s")
    t_u = [tu0, tu1, tu2, tu3]
    t_v = [tv0, tv1, tv2, tv3]
    base = s * TILE_ROWS

    _zero_rbuf(rbuf)

    def load_phase(tbl):
      with jax.named_scope("load_phase"):
        @pl.loop(0, RB_ITERS)
        def _zero(k):
            pltpu.sync_copy(rbuf, acc.at[pl.ds(base + k * RB, RB)])

        pltpu.sync_copy(tbl.at[pl.ds(base, TILE_ROWS)],
                        tbl_s.at[pl.ds(base, TILE_ROWS)])

    def edge_phase(p):
      with jax.named_scope("edge_phase"):
        bufs = (rows, rows_b, rows_c, rows_d)
        gsems = (gsem, gsem_b, gsem_c, gsem_d)
        ssems = (ssem, ssem_b, ssem_c, ssem_d)
        @pl.loop(0, CHUNKS // QCHUNKS)
        def _quarter(q):
            qs = pl.ds(q * QCHUNKS, QCHUNKS)

            @pl.when(c == 0)
            def _():
                pltpu.sync_copy(ev.at[p, s, qs], isrc)
                pltpu.sync_copy(eu.at[p, s, qs], idst)
                pltpu.sync_copy(val.at[p, s, qs], vbuf)

            @pl.when(c == 1)
            def _():
                pltpu.sync_copy(eu.at[p, s, qs], isrc)
                pltpu.sync_copy(ev.at[p, s, qs], idst)
                pltpu.sync_copy(valt.at[p, s, qs], vbuf)
            for b in range(2):
                pltpu.async_copy(tbl_s.at[isrc.at[b]], bufs[b], gsems[b])

            @pl.loop(0, QCHUNKS, step=4)
            def _chunk(j):
                for b in range(4):
                    jj = j + b
                    pltpu.make_async_copy(tbl_s.at[isrc.at[jj]], bufs[b],
                                          gsems[b]).wait()

                    @plsc.parallel_loop(0, CHUNK // 16)
                    def _scale(g):
                        vv = vbuf[jj, pl.ds(g * 16, 16)]
                        for t in range(16):
                            e = g * 16 + t
                            v = lax.gather(
                                vv, jnp.full((16, 1), t, jnp.int32),
                                lax.GatherDimensionNumbers(
                                    offset_dims=(), collapsed_slice_dims=(0,),
                                    start_index_map=(0,)),
                                slice_sizes=(1,),
                                mode=lax.GatherScatterMode.PROMISE_IN_BOUNDS)
                            for h in range(2):
                                sl = pl.ds(16 * h, 16)
                                bufs[b][e, sl] = bufs[b][e, sl] * v

                    pltpu.async_copy(bufs[b], acc.at[idst.at[jj]], ssems[b],
                                     add=True)

                    b2 = (b + 2) % 4

                    @pl.when(jj + 2 < QCHUNKS)
                    def _():
                        @pl.when(jj >= 2)
                        def _():
                            pltpu.make_async_copy(
                                bufs[b2], acc.at[idst.at[jj - 2]],
                                ssems[b2]).wait()

                        pltpu.async_copy(tbl_s.at[isrc.at[jj + 2]], bufs[b2],
                                         gsems[b2])

            for b in range(4):
                jj = QCHUNKS - 4 + b
                pltpu.make_async_copy(bufs[b], acc.at[idst.at[jj]],
                                      ssems[b]).wait()
            return None

    def store_phase(out_ref, col):
      with jax.named_scope("store_phase"):
        @pl.loop(0, RB_ITERS)
        def _store(k):
            r0 = base + k * RB
            pltpu.sync_copy(acc.at[pl.ds(r0, RB)], rbuf)

            @pl.loop(0, RB)
            def _relu(r):
                for h in range(2):
                    sl = pl.ds(16 * h, 16)
                    rbuf[r, sl] = jnp.maximum(rbuf[r, sl], 0.0)

            pltpu.sync_copy(rbuf, out_ref.at[pl.ds(r0, RB), pl.ds(col, DS)])

        _zero_rbuf(rbuf)

    for p in range(NSUP):
        @pl.when(c == 0)
        def _():
            load_phase(t_v[p])

        @pl.when(c == 1)
        def _():
            load_phase(t_u[p])

        plsc.subcore_barrier()

        edge_phase(p)

        plsc.subcore_barrier()

        @pl.when(c == 0)
        def _():
            store_phase(out_u, p * DS)

        @pl.when(c == 1)
        def _():
            store_phase(out_v, p * DS)


_SC_CALL_CACHE = []


def _sc_call(*args):
    if not _SC_CALL_CACHE:
        _SC_CALL_CACHE.append(pl.kernel(
            _sc_body,
            out_type=[jax.ShapeDtypeStruct((ACC_ROWS, D_OUT), jnp.float32)] * 2,
            mesh=plsc.VectorSubcoreMesh(core_axis_name="c", subcore_axis_name="s"),
            compiler_params=pltpu.CompilerParams(use_tc_tiling_on_sc=False),
            scratch_types=[
                pltpu.VMEM_SHARED((ACC_ROWS, DS), jnp.float32),
                pltpu.VMEM_SHARED((ACC_ROWS, DS), jnp.float32),
                pltpu.VMEM((CHUNK, DS), jnp.float32),
                pltpu.VMEM((CHUNK, DS), jnp.float32),
                pltpu.VMEM((CHUNK, DS), jnp.float32),
                pltpu.VMEM((CHUNK, DS), jnp.float32),
                pltpu.VMEM((RB, DS), jnp.float32),
                pltpu.VMEM((QCHUNKS, CHUNK), jnp.int32),
                pltpu.VMEM((QCHUNKS, CHUNK), jnp.int32),
                pltpu.VMEM((QCHUNKS, CHUNK), jnp.float32),
            ] + [pltpu.SemaphoreType.DMA] * 8,
        ))
    return _SC_CALL_CACHE[0](*args)


def kernel(x_u, x_v, edge_u, edge_v, edge_val, edge_val_t, W):
    xp_u = jnp.pad(x_u, ((0, ACC_ROWS - N_NODES), (0, 0)))
    xp_v = jnp.pad(x_v, ((0, ACC_ROWS - N_NODES), (0, 0)))
    tabs = _project(xp_u, xp_v, W)
    pad = E_PAD - E

    def pad4(a):
        return jnp.pad(a, ((0, 0), (0, pad))).reshape(NSUP, N_TILES, CHUNKS, CHUNK)

    eu = pad4(edge_u)
    ev = pad4(edge_v)
    val = pad4(edge_val)
    valt = pad4(edge_val_t)
    out_u, out_v = _sc_call(*tabs, eu, ev, val, valt)
    return out_u[:N_NODES], out_v[:N_NODES]


# speedup vs baseline: 1.4967x; 1.0390x over previous
import jax
import jax.numpy as jnp
from jax import lax
from jax.experimental import pallas as pl
from jax.experimental.pallas import tpu as pltpu
from jax.experimental.pallas import tpu_sc as plsc

N_NODES = 25000
D_IN = 128
D_OUT = 128
NSUP = 4
DS = D_OUT // NSUP
E = 160000
N_TILES = 16
CHUNK = 128
CHUNKS = 80
QCHUNKS = 20
E_PAD = N_TILES * CHUNKS * CHUNK
ACC_ROWS = 25088
TILE_ROWS = ACC_ROWS // N_TILES
RB = 112
RB_ITERS = TILE_ROWS // RB


def _project_body(xu_ref, xv_ref, w_ref, *out_refs):
    w = w_ref[...]
    hu = jnp.dot(xu_ref[...], w, preferred_element_type=jnp.float32)
    hv = jnp.dot(xv_ref[...], w, preferred_element_type=jnp.float32)
    for i in range(NSUP):
        out_refs[i][...] = hu[:, i * DS:(i + 1) * DS]
        out_refs[NSUP + i][...] = hv[:, i * DS:(i + 1) * DS]


def _project(x_u, x_v, W):
    rb = 1568
    return pl.pallas_call(
        _project_body,
        grid=(ACC_ROWS // rb,),
        in_specs=[
            pl.BlockSpec((rb, D_IN), lambda r: (r, 0)),
            pl.BlockSpec((rb, D_IN), lambda r: (r, 0)),
            pl.BlockSpec((D_IN, D_OUT), lambda r: (0, 0)),
        ],
        out_specs=[pl.BlockSpec((rb, DS), lambda r: (r, 0))] * (2 * NSUP),
        out_shape=[jax.ShapeDtypeStruct((ACC_ROWS, DS), jnp.float32)] * (2 * NSUP),
    )(x_u, x_v, W)


def _zero_rbuf(rbuf):
    @pl.loop(0, RB)
    def _z(r):
        for h in range(2):
            rbuf[r, pl.ds(16 * h, 16)] = jnp.zeros((16,), jnp.float32)


def _sc_body(tu0, tu1, tu2, tu3, tv0, tv1, tv2, tv3,
             eu, ev, val, valt, out_u, out_v,
             acc, tbl_s, rows, rows_b, rows_c, rows_d, rbuf, isrc, idst, vbuf,
             gsem, gsem_b, gsem_c, gsem_d, ssem, ssem_b, ssem_c, ssem_d):
    c = lax.axis_index("c")
    s = lax.axis_index("s")
    t_u = [tu0, tu1, tu2, tu3]
    t_v = [tv0, tv1, tv2, tv3]
    base = s * TILE_ROWS

    _zero_rbuf(rbuf)

    def load_phase(tbl):
      with jax.named_scope("load_phase"):
        @pl.loop(0, RB_ITERS)
        def _zero(k):
            pltpu.sync_copy(rbuf, acc.at[pl.ds(base + k * RB, RB)])

        pltpu.sync_copy(tbl.at[pl.ds(base, TILE_ROWS)],
                        tbl_s.at[pl.ds(base, TILE_ROWS)])

    def edge_phase(p):
      with jax.named_scope("edge_phase"):
        bufs = (rows, rows_b, rows_c, rows_d)
        gsems = (gsem, gsem_b, gsem_c, gsem_d)
        ssems = (ssem, ssem_b, ssem_c, ssem_d)
        @pl.loop(0, CHUNKS // QCHUNKS)
        def _quarter(q):
            qs = pl.ds(q * QCHUNKS, QCHUNKS)

            @pl.when(c == 0)
            def _():
                pltpu.sync_copy(ev.at[p, s, qs], isrc)
                pltpu.sync_copy(eu.at[p, s, qs], idst)
                pltpu.sync_copy(val.at[p, s, qs], vbuf)

            @pl.when(c == 1)
            def _():
                pltpu.sync_copy(eu.at[p, s, qs], isrc)
                pltpu.sync_copy(ev.at[p, s, qs], idst)
                pltpu.sync_copy(valt.at[p, s, qs], vbuf)
            for b in range(2):
                pltpu.async_copy(tbl_s.at[isrc.at[b]], bufs[b], gsems[b])

            @pl.loop(0, QCHUNKS, step=4)
            def _chunk(j):
                for b in range(4):
                    jj = j + b
                    pltpu.make_async_copy(tbl_s.at[isrc.at[jj]], bufs[b],
                                          gsems[b]).wait()

                    @plsc.parallel_loop(0, CHUNK // 16)
                    def _scale(g):
                        vv = vbuf[jj, pl.ds(g * 16, 16)]
                        for t in range(16):
                            e = g * 16 + t
                            v = lax.gather(
                                vv, jnp.full((16, 1), t, jnp.int32),
                                lax.GatherDimensionNumbers(
                                    offset_dims=(), collapsed_slice_dims=(0,),
                                    start_index_map=(0,)),
                                slice_sizes=(1,),
                                mode=lax.GatherScatterMode.PROMISE_IN_BOUNDS)
                            for h in range(2):
                                sl = pl.ds(16 * h, 16)
                                bufs[b][e, sl] = bufs[b][e, sl] * v

                    pltpu.async_copy(bufs[b], acc.at[idst.at[jj]], ssems[b],
                                     add=True)

                    b2 = (b + 2) % 4

                    @pl.when(jj + 2 < QCHUNKS)
                    def _():
                        @pl.when(jj >= 2)
                        def _():
                            pltpu.make_async_copy(
                                bufs[b2], acc.at[idst.at[jj - 2]],
                                ssems[b2]).wait()

                        pltpu.async_copy(tbl_s.at[isrc.at[jj + 2]], bufs[b2],
                                         gsems[b2])

            for b in range(4):
                jj = QCHUNKS - 4 + b
                pltpu.make_async_copy(bufs[b], acc.at[idst.at[jj]],
                                      ssems[b]).wait()
            return None

    def store_phase(out_ref, col):
      with jax.named_scope("store_phase"):
        def store_rb(r0, nr):
            pltpu.sync_copy(acc.at[pl.ds(r0, nr)], rbuf.at[pl.ds(0, nr)])

            @pl.loop(0, nr)
            def _relu(r):
                for h in range(2):
                    sl = pl.ds(16 * h, 16)
                    rbuf[r, sl] = jnp.maximum(rbuf[r, sl], 0.0)

            pltpu.sync_copy(rbuf.at[pl.ds(0, nr)],
                            out_ref.at[pl.ds(r0, nr), pl.ds(col, DS)])

        @pl.when(s < N_TILES - 1)
        def _():
            @pl.loop(0, RB_ITERS)
            def _store(k):
                store_rb(base + k * RB, RB)

        @pl.when(s == N_TILES - 1)
        def _():
            @pl.loop(0, RB_ITERS - 1)
            def _store(k):
                store_rb(base + k * RB, RB)

            store_rb(base + (RB_ITERS - 1) * RB, N_NODES - (N_TILES - 1) * TILE_ROWS - (RB_ITERS - 1) * RB)

        _zero_rbuf(rbuf)

    for p in range(NSUP):
        @pl.when(c == 0)
        def _():
            load_phase(t_v[p])

        @pl.when(c == 1)
        def _():
            load_phase(t_u[p])

        plsc.subcore_barrier()

        edge_phase(p)

        plsc.subcore_barrier()

        @pl.when(c == 0)
        def _():
            store_phase(out_u, p * DS)

        @pl.when(c == 1)
        def _():
            store_phase(out_v, p * DS)


_SC_CALL_CACHE = []


def _sc_call(*args):
    if not _SC_CALL_CACHE:
        _SC_CALL_CACHE.append(pl.kernel(
            _sc_body,
            out_type=[jax.ShapeDtypeStruct((N_NODES, D_OUT), jnp.float32)] * 2,
            mesh=plsc.VectorSubcoreMesh(core_axis_name="c", subcore_axis_name="s"),
            compiler_params=pltpu.CompilerParams(use_tc_tiling_on_sc=False),
            scratch_types=[
                pltpu.VMEM_SHARED((ACC_ROWS, DS), jnp.float32),
                pltpu.VMEM_SHARED((ACC_ROWS, DS), jnp.float32),
                pltpu.VMEM((CHUNK, DS), jnp.float32),
                pltpu.VMEM((CHUNK, DS), jnp.float32),
                pltpu.VMEM((CHUNK, DS), jnp.float32),
                pltpu.VMEM((CHUNK, DS), jnp.float32),
                pltpu.VMEM((RB, DS), jnp.float32),
                pltpu.VMEM((QCHUNKS, CHUNK), jnp.int32),
                pltpu.VMEM((QCHUNKS, CHUNK), jnp.int32),
                pltpu.VMEM((QCHUNKS, CHUNK), jnp.float32),
            ] + [pltpu.SemaphoreType.DMA] * 8,
        ))
    return _SC_CALL_CACHE[0](*args)


def kernel(x_u, x_v, edge_u, edge_v, edge_val, edge_val_t, W):
    xp_u = jnp.pad(x_u, ((0, ACC_ROWS - N_NODES), (0, 0)))
    xp_v = jnp.pad(x_v, ((0, ACC_ROWS - N_NODES), (0, 0)))
    tabs = _project(xp_u, xp_v, W)
    pad = E_PAD - E

    def pad4(a):
        return jnp.pad(a, ((0, 0), (0, pad))).reshape(NSUP, N_TILES, CHUNKS, CHUNK)

    eu = pad4(edge_u)
    ev = pad4(edge_v)
    val = pad4(edge_val)
    valt = pad4(edge_val_t)
    return _sc_call(*tabs, eu, ev, val, valt)


# speedup vs baseline: 1.4996x; 1.0020x over previous
import jax
import jax.numpy as jnp
from jax import lax
from jax.experimental import pallas as pl
from jax.experimental.pallas import tpu as pltpu
from jax.experimental.pallas import tpu_sc as plsc

N_NODES = 25000
D_IN = 128
D_OUT = 128
NSUP = 4
DS = D_OUT // NSUP
E = 160000
N_TILES = 16
CHUNK = 128
CHUNKS = 80
QCHUNKS = 20
E_PAD = N_TILES * CHUNKS * CHUNK
ACC_ROWS = 25088
TILE_ROWS = ACC_ROWS // N_TILES
RB = 112
RB_ITERS = TILE_ROWS // RB


def _project_body(xu_ref, xv_ref, w_ref, *out_refs):
    w = w_ref[...]
    hu = jnp.dot(xu_ref[...], w, preferred_element_type=jnp.float32)
    hv = jnp.dot(xv_ref[...], w, preferred_element_type=jnp.float32)
    for i in range(NSUP):
        out_refs[i][...] = hu[:, i * DS:(i + 1) * DS]
        out_refs[NSUP + i][...] = hv[:, i * DS:(i + 1) * DS]


def _project(x_u, x_v, W):
    rb = 1568
    return pl.pallas_call(
        _project_body,
        grid=(ACC_ROWS // rb,),
        in_specs=[
            pl.BlockSpec((rb, D_IN), lambda r: (r, 0)),
            pl.BlockSpec((rb, D_IN), lambda r: (r, 0)),
            pl.BlockSpec((D_IN, D_OUT), lambda r: (0, 0)),
        ],
        out_specs=[pl.BlockSpec((rb, DS), lambda r: (r, 0))] * (2 * NSUP),
        out_shape=[jax.ShapeDtypeStruct((ACC_ROWS, DS), jnp.float32)] * (2 * NSUP),
    )(x_u, x_v, W)


def _zero_rbuf(rbuf):
    @pl.loop(0, RB)
    def _z(r):
        for h in range(2):
            rbuf[r, pl.ds(16 * h, 16)] = jnp.zeros((16,), jnp.float32)


def _sc_body(tu0, tu1, tu2, tu3, tv0, tv1, tv2, tv3,
             eu, ev, val, valt, out_u, out_v,
             acc, tbl_s, rows, rows_b, rows_c, rows_d, rbuf, isrc, idst, vbuf,
             gsem, gsem_b, gsem_c, gsem_d, ssem, ssem_b, ssem_c, ssem_d):
    c = lax.axis_index("c")
    s = lax.axis_index("s")
    t_u = [tu0, tu1, tu2, tu3]
    t_v = [tv0, tv1, tv2, tv3]
    base = s * TILE_ROWS

    _zero_rbuf(rbuf)

    def load_phase(tbl):
      with jax.named_scope("load_phase"):
        @pl.loop(0, RB_ITERS)
        def _zero(k):
            pltpu.sync_copy(rbuf, acc.at[pl.ds(base + k * RB, RB)])

        pltpu.sync_copy(tbl.at[pl.ds(base, TILE_ROWS)],
                        tbl_s.at[pl.ds(base, TILE_ROWS)])

    def edge_phase(p):
      with jax.named_scope("edge_phase"):
        bufs = (rows, rows_b, rows_c, rows_d)
        gsems = (gsem, gsem_b, gsem_c, gsem_d)
        ssems = (ssem, ssem_b, ssem_c, ssem_d)
        @pl.loop(0, CHUNKS // QCHUNKS)
        def _quarter(q):
            qs = pl.ds(q * QCHUNKS, QCHUNKS)

            @pl.when(c == 0)
            def _():
                pltpu.sync_copy(ev.at[p, s, qs], isrc)
                pltpu.sync_copy(eu.at[p, s, qs], idst)
                pltpu.sync_copy(val.at[p, s, qs], vbuf)

            @pl.when(c == 1)
            def _():
                pltpu.sync_copy(eu.at[p, s, qs], isrc)
                pltpu.sync_copy(ev.at[p, s, qs], idst)
                pltpu.sync_copy(valt.at[p, s, qs], vbuf)
            for b in range(2):
                pltpu.async_copy(tbl_s.at[isrc.at[b]], bufs[b], gsems[b])

            @pl.loop(0, QCHUNKS, step=4)
            def _chunk(j):
                for b in range(4):
                    jj = j + b
                    pltpu.make_async_copy(tbl_s.at[isrc.at[jj]], bufs[b],
                                          gsems[b]).wait()

                    @plsc.parallel_loop(0, CHUNK // 16)
                    def _scale(g):
                        vv = vbuf[jj, pl.ds(g * 16, 16)]
                        for t in range(16):
                            e = g * 16 + t
                            v = lax.gather(
                                vv, jnp.full((16, 1), t, jnp.int32),
                                lax.GatherDimensionNumbers(
                                    offset_dims=(), collapsed_slice_dims=(0,),
                                    start_index_map=(0,)),
                                slice_sizes=(1,),
                                mode=lax.GatherScatterMode.PROMISE_IN_BOUNDS)
                            for h in range(2):
                                sl = pl.ds(16 * h, 16)
                                bufs[b][e, sl] = bufs[b][e, sl] * v

                    pltpu.async_copy(bufs[b], acc.at[idst.at[jj]], ssems[b],
                                     add=True)

                    b2 = (b + 2) % 4

                    @pl.when(jj + 2 < QCHUNKS)
                    def _():
                        @pl.when(jj >= 2)
                        def _():
                            pltpu.make_async_copy(
                                bufs[b2], acc.at[idst.at[jj - 2]],
                                ssems[b2]).wait()

                        pltpu.async_copy(tbl_s.at[isrc.at[jj + 2]], bufs[b2],
                                         gsems[b2])

            for b in range(4):
                jj = QCHUNKS - 4 + b
                pltpu.make_async_copy(bufs[b], acc.at[idst.at[jj]],
                                      ssems[b]).wait()
            return None

    def store_phase(out_ref, col):
      with jax.named_scope("store_phase"):
        def store_rb(r0, nr):
            pltpu.sync_copy(acc.at[pl.ds(r0, nr)], rbuf.at[pl.ds(0, nr)])

            @pl.loop(0, nr)
            def _relu(r):
                for h in range(2):
                    sl = pl.ds(16 * h, 16)
                    rbuf[r, sl] = jnp.maximum(rbuf[r, sl], 0.0)

            pltpu.sync_copy(rbuf.at[pl.ds(0, nr)],
                            out_ref.at[pl.ds(r0, nr), pl.ds(col, DS)])

        @pl.when(s < N_TILES - 1)
        def _():
            @pl.loop(0, RB_ITERS)
            def _store(k):
                store_rb(base + k * RB, RB)

        @pl.when(s == N_TILES - 1)
        def _():
            @pl.loop(0, RB_ITERS - 1)
            def _store(k):
                store_rb(base + k * RB, RB)

            store_rb(base + (RB_ITERS - 1) * RB, N_NODES - (N_TILES - 1) * TILE_ROWS - (RB_ITERS - 1) * RB)

        _zero_rbuf(rbuf)

    for p in range(NSUP):
        @pl.when(c == 0)
        def _():
            load_phase(t_v[p])

        @pl.when(c == 1)
        def _():
            load_phase(t_u[p])

        plsc.subcore_barrier()

        edge_phase(p)

        plsc.subcore_barrier()

        @pl.when(c == 0)
        def _():
            store_phase(out_u, p * DS)

        @pl.when(c == 1)
        def _():
            store_phase(out_v, p * DS)


_SC_CALL_CACHE = []


def _sc_call(*args):
    if not _SC_CALL_CACHE:
        _SC_CALL_CACHE.append(pl.kernel(
            _sc_body,
            out_type=[jax.ShapeDtypeStruct((N_NODES, D_OUT), jnp.float32)] * 2,
            mesh=plsc.VectorSubcoreMesh(core_axis_name="c", subcore_axis_name="s"),
            compiler_params=pltpu.CompilerParams(use_tc_tiling_on_sc=False),
            scratch_types=[
                pltpu.VMEM_SHARED((ACC_ROWS, DS), jnp.float32),
                pltpu.VMEM_SHARED((ACC_ROWS, DS), jnp.float32),
                pltpu.VMEM((CHUNK, DS), jnp.float32),
                pltpu.VMEM((CHUNK, DS), jnp.float32),
                pltpu.VMEM((CHUNK, DS), jnp.float32),
                pltpu.VMEM((CHUNK, DS), jnp.float32),
                pltpu.VMEM((RB, DS), jnp.float32),
                pltpu.VMEM((QCHUNKS, CHUNK), jnp.int32),
                pltpu.VMEM((QCHUNKS, CHUNK), jnp.int32),
                pltpu.VMEM((QCHUNKS, CHUNK), jnp.float32),
            ] + [pltpu.SemaphoreType.DMA] * 8,
        ))
    return _SC_CALL_CACHE[0](*args)


def kernel(x_u, x_v, edge_u, edge_v, edge_val, edge_val_t, W):
    xp_u = jnp.pad(x_u, ((0, ACC_ROWS - N_NODES), (0, 0)))
    xp_v = jnp.pad(x_v, ((0, ACC_ROWS - N_NODES), (0, 0)))
    tabs = _project(xp_u, xp_v, W)
    pad = E_PAD - E

    def pad4(a):
        return jnp.pad(a, ((0, 0), (0, pad))).reshape(NSUP, N_TILES, CHUNKS, CHUNK)

    eu = pad4(edge_u)
    ev = pad4(edge_v)
    val = pad4(edge_val)
    valt = pad4(edge_val_t)
    out_u, out_v = _sc_call(*tabs, eu, ev, val, valt)
    return out_u, out_v
